# baseline (speedup 1.0000x reference)
import functools

import jax
import jax.numpy as jnp
from jax import lax
from jax.experimental import pallas as pl
from jax.experimental.pallas import tpu as pltpu
from jax.experimental.pallas import tpu_sc as plsc

N = 10000
E = 320000
D = 128
GAMMA = 0.2

NC = 2
NS = 16
NW = NC * NS
CPT = E // NW
CH = 80
NCH = CPT // CH
RPT = 640

_mesh = plsc.VectorSubcoreMesh(
    core_axis_name="c", subcore_axis_name="s", num_cores=NC, num_subcores=NS)
_sc_params = pltpu.CompilerParams(
    use_tc_tiling_on_sc=False, needs_layout_passes=False)


def _proj(h, w_ref, a_ref):
    u = lax.dot_general(a_ref[0:1, 0:D], w_ref[...], (((1,), (0,)), ((), ())),
                        preferred_element_type=jnp.float32)
    v = lax.dot_general(a_ref[0:1, D:2 * D], w_ref[...],
                        (((1,), (0,)), ((), ())),
                        preferred_element_type=jnp.float32)
    s = jnp.sum(h * u, axis=1)
    t = jnp.sum(h * v, axis=1)
    return s, t


def _project_body(h_ref, w_ref, a_ref, s_ref, t_ref):
    s, t = _proj(h_ref[...], w_ref, a_ref)
    s_ref[...] = s
    t_ref[...] = t


def _tc_project(h, W_fc, a_attn):
    return pl.pallas_call(
        _project_body,
        out_shape=[jax.ShapeDtypeStruct((N,), jnp.float32),
                   jax.ShapeDtypeStruct((N,), jnp.float32)],
    )(h, W_fc, a_attn)


def _den_col(den_ref):
    ones = jnp.ones((2, 1), jnp.float32)
    dsum = lax.dot_general(den_ref[...], ones, (((0,), (0,)), ((), ())),
                           preferred_element_type=jnp.float32)
    return 1.0 / jnp.maximum(dsum, 1e-16)


def _combine_proj_body(emb_ref, agg_ref, den_ref, w_ref, a_ref,
                       h_ref, s_ref, t_ref):
    dinv = _den_col(den_ref)
    hagg = (agg_ref[0] + agg_ref[1]) * dinv
    h = 0.5 * (emb_ref[...] + hagg)
    h_ref[...] = h
    s, t = _proj(h, w_ref, a_ref)
    s_ref[...] = s
    t_ref[...] = t


def _tc_combine_proj(emb, agg, den, W_fc, a_attn):
    return pl.pallas_call(
        _combine_proj_body,
        out_shape=[jax.ShapeDtypeStruct((N, D), jnp.float32),
                   jax.ShapeDtypeStruct((N,), jnp.float32),
                   jax.ShapeDtypeStruct((N,), jnp.float32)],
    )(emb, agg, den, W_fc, a_attn)


def _combine_loss_body(emb_ref, agg_ref, den_ref, h_ref, hb_ref, la_ref):
    dinv = _den_col(den_ref)
    hagg = (agg_ref[0] + agg_ref[1]) * dinv
    h = 0.5 * (emb_ref[...] + hagg)
    h_ref[...] = h
    hb_ref[...] = h.astype(jnp.bfloat16)
    diff = h - emb_ref[...]
    la_ref[...] = jnp.reshape(jnp.sum(diff * diff), (1, 1))


def _tc_combine_loss(emb, agg, den):
    return pl.pallas_call(
        _combine_loss_body,
        out_shape=[jax.ShapeDtypeStruct((N, D), jnp.float32),
                   jax.ShapeDtypeStruct((N, D), jnp.bfloat16),
                   jax.ShapeDtypeStruct((1, 1), jnp.float32)],
    )(emb, agg, den)


def _loss_final_body(attn_ref, d2_ref, la_ref, out_ref):
    dist = jnp.sqrt(d2_ref[...] + 1e-12)
    lb = jnp.sum(attn_ref[...] * dist)
    out_ref[...] = (la_ref[...] + lb) / N


def _tc_loss_final(attn, d2, la):
    return pl.pallas_call(
        _loss_final_body,
        out_shape=jax.ShapeDtypeStruct((1, 1), jnp.float32),
    )(attn, d2, la)


def _sc_layer_body(h_hbm, ei_hbm, s_hbm, t_hbm,
                   zn_hbm,
                   agg_out, den_out, w_out,
                   src_v, dst2d,
                   sg0, sg1, tg0, tg1, w0, w1, rows0, rows1,
                   agg_sh, den_sh,
                   gs0, gs1, gt0, gt1, gr0, gr1, ds0, ds1, os0, os1,
                   ss0, ss1):
    cid = lax.axis_index("c")
    sid = lax.axis_index("s")
    wid = cid * NS + sid
    base = wid * CPT

    sg = (sg0, sg1)
    tg = (tg0, tg1)
    wv = (w0, w1)
    rows = (rows0, rows1)
    gssem = (gs0, gs1)
    gtsem = (gt0, gt1)
    grsem = (gr0, gr1)
    dsem = (ds0, ds1)
    osem = (os0, os1)
    ssem = (ss0, ss1)

    def zbody(r, c):
        for k in range(D // 16):
            rows0[r, pl.ds(k * 16, 16)] = jnp.zeros((16,), jnp.float32)
        return c

    lax.fori_loop(0, CH, zbody, 0)
    for b in range(RPT // CH):
        @pl.when(sid < 15)
        def _():
            pltpu.sync_copy(rows0,
                            agg_sh.at[pl.ds(sid * RPT + b * CH, CH)])
    for b in range((N - 15 * RPT) // CH):
        @pl.when(sid == 15)
        def _():
            pltpu.sync_copy(rows0,
                            agg_sh.at[pl.ds(15 * RPT + b * CH, CH)])

    @pl.when(sid == 0)
    def _():
        pltpu.sync_copy(zn_hbm, den_sh)

    pltpu.sync_copy(ei_hbm.at[1, pl.ds(base, CPT)], src_v)

    def dbody(j, carry):
        for k in range(CH // 16):
            dst2d[j, pl.ds(k * 16, 16)] = src_v[pl.ds(j * CH + k * 16, 16)]
        return carry

    lax.fori_loop(0, NCH, dbody, 0)
    pltpu.sync_copy(ei_hbm.at[0, pl.ds(base, CPT)], src_v)

    plsc.subcore_barrier()

    def issue_gathers(jj, q):
        csl = pl.ds(jj * CH, CH)
        pltpu.async_copy(s_hbm.at[src_v.at[csl]], sg[q], gssem[q])
        pltpu.async_copy(t_hbm.at[dst2d.at[jj]], tg[q], gtsem[q])
        pltpu.async_copy(h_hbm.at[src_v.at[csl]], rows[q], grsem[q])

    def step(j, p, wait_prev_scatter, issue_next, wait_prev2_w):
        q = 1 - p
        if wait_prev_scatter:
            pltpu.make_async_copy(
                rows[q], agg_sh.at[dst2d.at[j - 1]], ssem[q]).wait()
        if issue_next:
            issue_gathers(j + 1, q)
        csl = pl.ds(j * CH, CH)
        pltpu.make_async_copy(s_hbm.at[src_v.at[csl]], sg[p], gssem[p]).wait()
        pltpu.make_async_copy(t_hbm.at[dst2d.at[j]], tg[p], gtsem[p]).wait()
        pltpu.make_async_copy(h_hbm.at[src_v.at[csl]], rows[p],
                              grsem[p]).wait()
        if wait_prev2_w:
            pltpu.make_async_copy(
                wv[p], den_sh.at[dst2d.at[j - 2]], dsem[p]).wait()
            pltpu.make_async_copy(
                wv[p], w_out.at[pl.ds(base + (j - 2) * CH, CH)],
                osem[p]).wait()
        for k in range(CH // 16):
            ks = pl.ds(k * 16, 16)
            e = sg[p][ks] + tg[p][ks]
            e = jnp.where(e >= 0.0, e, GAMMA * e)
            wv[p][ks] = jnp.exp(e)
        pltpu.async_copy(wv[p], den_sh.at[dst2d.at[j]], dsem[p], add=True)
        pltpu.async_copy(wv[p], w_out.at[pl.ds(base + j * CH, CH)], osem[p])

        def rbody(g2, c2):
            w16 = wv[p][pl.ds(g2 * 16, 16)]
            for k in range(16):
                r = g2 * 16 + k
                wsc = w16[k]
                for u in range(D // 16):
                    qs = pl.ds(u * 16, 16)
                    rows[p][r, qs] = rows[p][r, qs] * wsc
            return c2

        lax.fori_loop(0, CH // 16, rbody, 0)
        pltpu.async_copy(rows[p], agg_sh.at[dst2d.at[j]], ssem[p], add=True)

    issue_gathers(0, 0)
    step(0, 0, wait_prev_scatter=False, issue_next=True, wait_prev2_w=False)
    step(1, 1, wait_prev_scatter=True, issue_next=True, wait_prev2_w=False)

    def pair(k2, carry):
        j0 = 2 * k2 + 2
        step(j0, 0, wait_prev_scatter=True, issue_next=True,
             wait_prev2_w=True)
        step(j0 + 1, 1, wait_prev_scatter=True, issue_next=True,
             wait_prev2_w=True)
        return carry

    lax.fori_loop(0, (NCH - 3) // 2, pair, 0)
    step(NCH - 1, 0, wait_prev_scatter=True, issue_next=False,
         wait_prev2_w=True)

    jl = NCH - 1
    pltpu.make_async_copy(rows[0], agg_sh.at[dst2d.at[jl]], ssem[0]).wait()
    pltpu.make_async_copy(wv[0], den_sh.at[dst2d.at[jl]], dsem[0]).wait()
    pltpu.make_async_copy(
        wv[0], w_out.at[pl.ds(base + jl * CH, CH)], osem[0]).wait()
    pltpu.make_async_copy(wv[1], den_sh.at[dst2d.at[jl - 1]], dsem[1]).wait()
    pltpu.make_async_copy(
        wv[1], w_out.at[pl.ds(base + (jl - 1) * CH, CH)], osem[1]).wait()

    plsc.subcore_barrier()

    rstripe = pl.ds(sid * RPT, RPT)
    rtail = pl.ds(15 * RPT, N - 15 * RPT)

    @pl.when(sid < 15)
    def _():
        pltpu.sync_copy(agg_sh.at[rstripe], agg_out.at[cid, rstripe])

    @pl.when(sid == 15)
    def _():
        pltpu.sync_copy(agg_sh.at[rtail], agg_out.at[cid, rtail])

    @pl.when(sid == 0)
    def _():
        pltpu.sync_copy(den_sh, den_out.at[cid])


def _sc_layer(h, ei, s, t, zn):
    f = pl.kernel(
        _sc_layer_body,
        out_type=[jax.ShapeDtypeStruct((NC, N, D), jnp.float32),
                  jax.ShapeDtypeStruct((NC, N), jnp.float32),
                  jax.ShapeDtypeStruct((E,), jnp.float32)],
        mesh=_mesh,
        scratch_types=[
            pltpu.VMEM((CPT,), jnp.int32),
            pltpu.VMEM((NCH, CH), jnp.int32),
            pltpu.VMEM((CH,), jnp.float32),
            pltpu.VMEM((CH,), jnp.float32),
            pltpu.VMEM((CH,), jnp.float32),
            pltpu.VMEM((CH,), jnp.float32),
            pltpu.VMEM((CH,), jnp.float32),
            pltpu.VMEM((CH,), jnp.float32),
            pltpu.VMEM((CH, D), jnp.float32),
            pltpu.VMEM((CH, D), jnp.float32),
            pltpu.VMEM_SHARED((N, D), jnp.float32),
            pltpu.VMEM_SHARED((N,), jnp.float32),
        ] + [pltpu.SemaphoreType.DMA] * 12,
        compiler_params=_sc_params,
    )
    return f(h, ei, s, t, zn)


def _sc_loss_body(h_hbm, ei_hbm, w_hbm, den_hbm,
                  attn_out, d2_out,
                  src_v, dst_v, w_v, den0_v, den1_v, attn_v, d2_v, tacc,
                  ra0, ra1, rb0, rb1,
                  ga0, ga1, gb0, gb1):
    cid = lax.axis_index("c")
    sid = lax.axis_index("s")
    wid = cid * NS + sid
    base = wid * CPT

    rows_a = (ra0, ra1)
    rows_b = (rb0, rb1)
    gasem = (ga0, ga1)
    gbsem = (gb0, gb1)

    pltpu.sync_copy(den_hbm.at[0], den0_v)
    pltpu.sync_copy(den_hbm.at[1], den1_v)
    pltpu.sync_copy(w_hbm.at[pl.ds(base, CPT)], w_v)
    pltpu.sync_copy(ei_hbm.at[0, pl.ds(base, CPT)], src_v)
    pltpu.sync_copy(ei_hbm.at[1, pl.ds(base, CPT)], dst_v)

    def sbody(g, carry):
        sl = pl.ds(g * 16, 16)
        d16 = dst_v[sl]
        dg = plsc.load_gather(den0_v, [d16]) + plsc.load_gather(den1_v, [d16])
        attn_v[sl] = w_v[sl] / jnp.maximum(dg, 1e-16)
        return carry

    lax.fori_loop(0, CPT // 16, sbody, 0)
    pltpu.sync_copy(attn_v, attn_out.at[wid])

    def issue(jj, q):
        csl = pl.ds(jj * CH, CH)
        pltpu.async_copy(h_hbm.at[src_v.at[csl]], rows_a[q], gasem[q])
        pltpu.async_copy(h_hbm.at[dst_v.at[csl]], rows_b[q], gbsem[q])

    def step(j, p, issue_next):
        q = 1 - p
        if issue_next:
            @pl.when(j + 1 < NCH)
            def _():
                issue(j + 1, q)
        csl = pl.ds(j * CH, CH)
        pltpu.make_async_copy(h_hbm.at[src_v.at[csl]], rows_a[p],
                              gasem[p]).wait()
        pltpu.make_async_copy(h_hbm.at[dst_v.at[csl]], rows_b[p],
                              gbsem[p]).wait()

        def rbody(g2, c2):
            lanes = lax.iota(jnp.int32, 16)
            for k in range(16):
                r = g2 * 16 + k
                acc32 = jnp.zeros((32,), jnp.bfloat16)
                for u in range(D // 32):
                    qs = pl.ds(u * 32, 32)
                    dd = rows_a[p][r, qs] - rows_b[p][r, qs]
                    acc32 = acc32 + dd * dd
                lo, hi = plsc.unpack(
                    acc32, format=plsc.PackFormat.INTERLEAVED,
                    preferred_element_type=jnp.float32)
                tacc[k, pl.ds(0, 16)] = lo + hi
            d2vec = jnp.zeros((16,), jnp.float32)
            for i in range(16):
                col = jnp.full((16,), i, jnp.int32)
                d2vec = d2vec + plsc.load_gather(tacc, [lanes, col])
            d2_v[pl.ds(j * CH + g2 * 16, 16)] = d2vec
            return c2

        lax.fori_loop(0, CH // 16, rbody, 0)

    issue(0, 0)
    step(0, 0, issue_next=True)

    def pair(k2, carry):
        j0 = 2 * k2 + 1
        step(j0, 1, issue_next=True)
        step(j0 + 1, 0, issue_next=True)
        return carry

    lax.fori_loop(0, (NCH - 1) // 2, pair, 0)
    pltpu.sync_copy(d2_v, d2_out.at[wid])


def _sc_loss(h2, ei, w, den):
    f = pl.kernel(
        _sc_loss_body,
        out_type=[jax.ShapeDtypeStruct((NW, CPT), jnp.float32),
                  jax.ShapeDtypeStruct((NW, CPT), jnp.float32)],
        mesh=_mesh,
        scratch_types=[
            pltpu.VMEM((CPT,), jnp.int32),
            pltpu.VMEM((CPT,), jnp.int32),
            pltpu.VMEM((CPT,), jnp.float32),
            pltpu.VMEM((N,), jnp.float32),
            pltpu.VMEM((N,), jnp.float32),
            pltpu.VMEM((CPT,), jnp.float32),
            pltpu.VMEM((CPT,), jnp.float32),
            pltpu.VMEM((16, 16), jnp.float32),
            pltpu.VMEM((CH, D), jnp.bfloat16),
            pltpu.VMEM((CH, D), jnp.bfloat16),
            pltpu.VMEM((CH, D), jnp.bfloat16),
            pltpu.VMEM((CH, D), jnp.bfloat16),
        ] + [pltpu.SemaphoreType.DMA] * 4,
        compiler_params=_sc_params,
    )
    return f(h2, ei, w, den)


def kernel(embedding_input, edge_index, W_fc, a_attn):
    emb = embedding_input
    zn = jnp.zeros((N,), jnp.float32)

    s1, t1 = _tc_project(emb, W_fc, a_attn)
    agg1, den1, _ = _sc_layer(emb, edge_index, s1, t1, zn)
    h1, s2, t2 = _tc_combine_proj(emb, agg1, den1, W_fc, a_attn)

    agg2, den2, w2 = _sc_layer(h1, edge_index, s2, t2, zn)
    h2, h2b, la = _tc_combine_loss(emb, agg2, den2)

    attn, d2 = _sc_loss(h2b, edge_index, w2, den2)
    loss = _tc_loss_final(attn, d2, la)
    return h2, loss[0, 0]


# speedup vs baseline: 1.1200x; 1.1200x over previous
import functools

import jax
import jax.numpy as jnp
from jax import lax
from jax.experimental import pallas as pl
from jax.experimental.pallas import tpu as pltpu
from jax.experimental.pallas import tpu_sc as plsc

N = 10000
E = 320000
D = 128
GAMMA = 0.2

NC = 2
NS = 16
NW = NC * NS
CPT = E // NW
CH = 80
NCH = CPT // CH
RPT = 640

_mesh = plsc.VectorSubcoreMesh(
    core_axis_name="c", subcore_axis_name="s", num_cores=NC, num_subcores=NS)
_sc_params = pltpu.CompilerParams(
    use_tc_tiling_on_sc=False, needs_layout_passes=False)


def _proj(h, w_ref, a_ref):
    u = lax.dot_general(a_ref[0:1, 0:D], w_ref[...], (((1,), (0,)), ((), ())),
                        preferred_element_type=jnp.float32)
    v = lax.dot_general(a_ref[0:1, D:2 * D], w_ref[...],
                        (((1,), (0,)), ((), ())),
                        preferred_element_type=jnp.float32)
    s = jnp.sum(h * u, axis=1)
    t = jnp.sum(h * v, axis=1)
    return s, t


def _project_body(h_ref, w_ref, a_ref, s_ref, t_ref):
    s, t = _proj(h_ref[...], w_ref, a_ref)
    s_ref[...] = s
    t_ref[...] = t


def _tc_project(h, W_fc, a_attn):
    return pl.pallas_call(
        _project_body,
        out_shape=[jax.ShapeDtypeStruct((N,), jnp.float32),
                   jax.ShapeDtypeStruct((N,), jnp.float32)],
    )(h, W_fc, a_attn)


def _den_col(den_ref):
    ones = jnp.ones((2, 1), jnp.float32)
    dsum = lax.dot_general(den_ref[...], ones, (((0,), (0,)), ((), ())),
                           preferred_element_type=jnp.float32)
    return 1.0 / jnp.maximum(dsum, 1e-16)


def _combine_proj_body(emb_ref, agg_ref, den_ref, w_ref, a_ref,
                       h_ref, s_ref, t_ref):
    dinv = _den_col(den_ref)
    hagg = (agg_ref[0] + agg_ref[1]) * dinv
    h = 0.5 * (emb_ref[...] + hagg)
    h_ref[...] = h
    s, t = _proj(h, w_ref, a_ref)
    s_ref[...] = s
    t_ref[...] = t


def _tc_combine_proj(emb, agg, den, W_fc, a_attn):
    return pl.pallas_call(
        _combine_proj_body,
        out_shape=[jax.ShapeDtypeStruct((N, D), jnp.float32),
                   jax.ShapeDtypeStruct((N,), jnp.float32),
                   jax.ShapeDtypeStruct((N,), jnp.float32)],
    )(emb, agg, den, W_fc, a_attn)


def _combine_loss_body(emb_ref, agg_ref, den_ref, h_ref, hb_ref, la_ref):
    dinv = _den_col(den_ref)
    hagg = (agg_ref[0] + agg_ref[1]) * dinv
    h = 0.5 * (emb_ref[...] + hagg)
    h_ref[...] = h
    hb_ref[...] = h.astype(jnp.bfloat16)
    diff = h - emb_ref[...]
    la_ref[...] = jnp.reshape(jnp.sum(diff * diff), (1, 1))


def _tc_combine_loss(emb, agg, den):
    return pl.pallas_call(
        _combine_loss_body,
        out_shape=[jax.ShapeDtypeStruct((N, D), jnp.float32),
                   jax.ShapeDtypeStruct((N, D), jnp.bfloat16),
                   jax.ShapeDtypeStruct((1, 1), jnp.float32)],
    )(emb, agg, den)


def _loss_final_body(attn_ref, d2_ref, la_ref, out_ref):
    dist = jnp.sqrt(d2_ref[...] + 1e-12)
    lb = jnp.sum(attn_ref[...] * dist)
    out_ref[...] = (la_ref[...] + lb) / N


def _tc_loss_final(attn, d2, la):
    return pl.pallas_call(
        _loss_final_body,
        out_shape=jax.ShapeDtypeStruct((1, 1), jnp.float32),
    )(attn, d2, la)


def _sc_layer_body(h_hbm, ei_hbm, s_hbm, t_hbm,
                   zn_hbm,
                   agg_out, den_out, w_out,
                   src_v, dst2d,
                   sg0, sg1, tg0, tg1, w0, w1, rows0, rows1,
                   agg_sh, den_sh,
                   gs0, gs1, gt0, gt1, gr0, gr1, ds0, ds1, os0, os1,
                   ss0, ss1):
    cid = lax.axis_index("c")
    sid = lax.axis_index("s")
    wid = cid * NS + sid
    base = wid * CPT

    sg = (sg0, sg1)
    tg = (tg0, tg1)
    wv = (w0, w1)
    rows = (rows0, rows1)
    gssem = (gs0, gs1)
    gtsem = (gt0, gt1)
    grsem = (gr0, gr1)
    dsem = (ds0, ds1)
    osem = (os0, os1)
    ssem = (ss0, ss1)

    def zbody(r, c):
        for k in range(D // 16):
            rows0[r, pl.ds(k * 16, 16)] = jnp.zeros((16,), jnp.float32)
        return c

    lax.fori_loop(0, CH, zbody, 0)
    for b in range(RPT // CH):
        @pl.when(sid < 15)
        def _():
            pltpu.sync_copy(rows0,
                            agg_sh.at[pl.ds(sid * RPT + b * CH, CH)])
    for b in range((N - 15 * RPT) // CH):
        @pl.when(sid == 15)
        def _():
            pltpu.sync_copy(rows0,
                            agg_sh.at[pl.ds(15 * RPT + b * CH, CH)])

    @pl.when(sid == 0)
    def _():
        pltpu.sync_copy(zn_hbm, den_sh)

    pltpu.sync_copy(ei_hbm.at[1, pl.ds(base, CPT)], src_v)

    def dbody(j, carry):
        for k in range(CH // 16):
            dst2d[j, pl.ds(k * 16, 16)] = src_v[pl.ds(j * CH + k * 16, 16)]
        return carry

    lax.fori_loop(0, NCH, dbody, 0)
    pltpu.sync_copy(ei_hbm.at[0, pl.ds(base, CPT)], src_v)

    plsc.subcore_barrier()

    def issue_gathers(jj, q):
        csl = pl.ds(jj * CH, CH)
        pltpu.async_copy(s_hbm.at[src_v.at[csl]], sg[q], gssem[q])
        pltpu.async_copy(t_hbm.at[dst2d.at[jj]], tg[q], gtsem[q])
        pltpu.async_copy(h_hbm.at[src_v.at[csl]], rows[q], grsem[q])

    def step(j, p, wait_prev_scatter, issue_next, wait_prev2_w):
        q = 1 - p
        if wait_prev_scatter:
            pltpu.make_async_copy(
                rows[q], agg_sh.at[dst2d.at[j - 1]], ssem[q]).wait()
        if issue_next:
            issue_gathers(j + 1, q)
        csl = pl.ds(j * CH, CH)
        pltpu.make_async_copy(s_hbm.at[src_v.at[csl]], sg[p], gssem[p]).wait()
        pltpu.make_async_copy(t_hbm.at[dst2d.at[j]], tg[p], gtsem[p]).wait()
        pltpu.make_async_copy(h_hbm.at[src_v.at[csl]], rows[p],
                              grsem[p]).wait()
        if wait_prev2_w:
            pltpu.make_async_copy(
                wv[p], den_sh.at[dst2d.at[j - 2]], dsem[p]).wait()
            pltpu.make_async_copy(
                wv[p], w_out.at[pl.ds(base + (j - 2) * CH, CH)],
                osem[p]).wait()
        for k in range(CH // 16):
            ks = pl.ds(k * 16, 16)
            e = sg[p][ks] + tg[p][ks]
            e = jnp.where(e >= 0.0, e, GAMMA * e)
            wv[p][ks] = jnp.exp(e)
        pltpu.async_copy(wv[p], den_sh.at[dst2d.at[j]], dsem[p], add=True)
        pltpu.async_copy(wv[p], w_out.at[pl.ds(base + j * CH, CH)], osem[p])

        def rbody(g2, c2):
            w16 = wv[p][pl.ds(g2 * 16, 16)]
            for k in range(16):
                r = g2 * 16 + k
                wsc = w16[k]
                for u in range(D // 16):
                    qs = pl.ds(u * 16, 16)
                    rows[p][r, qs] = rows[p][r, qs] * wsc
            return c2

        lax.fori_loop(0, CH // 16, rbody, 0)
        pltpu.async_copy(rows[p], agg_sh.at[dst2d.at[j]], ssem[p], add=True)

    issue_gathers(0, 0)
    step(0, 0, wait_prev_scatter=False, issue_next=True, wait_prev2_w=False)
    step(1, 1, wait_prev_scatter=True, issue_next=True, wait_prev2_w=False)

    def pair(k2, carry):
        j0 = 2 * k2 + 2
        step(j0, 0, wait_prev_scatter=True, issue_next=True,
             wait_prev2_w=True)
        step(j0 + 1, 1, wait_prev_scatter=True, issue_next=True,
             wait_prev2_w=True)
        return carry

    lax.fori_loop(0, (NCH - 3) // 2, pair, 0)
    step(NCH - 1, 0, wait_prev_scatter=True, issue_next=False,
         wait_prev2_w=True)

    jl = NCH - 1
    pltpu.make_async_copy(rows[0], agg_sh.at[dst2d.at[jl]], ssem[0]).wait()
    pltpu.make_async_copy(wv[0], den_sh.at[dst2d.at[jl]], dsem[0]).wait()
    pltpu.make_async_copy(
        wv[0], w_out.at[pl.ds(base + jl * CH, CH)], osem[0]).wait()
    pltpu.make_async_copy(wv[1], den_sh.at[dst2d.at[jl - 1]], dsem[1]).wait()
    pltpu.make_async_copy(
        wv[1], w_out.at[pl.ds(base + (jl - 1) * CH, CH)], osem[1]).wait()

    plsc.subcore_barrier()

    rstripe = pl.ds(sid * RPT, RPT)
    rtail = pl.ds(15 * RPT, N - 15 * RPT)

    @pl.when(sid < 15)
    def _():
        pltpu.sync_copy(agg_sh.at[rstripe], agg_out.at[cid, rstripe])

    @pl.when(sid == 15)
    def _():
        pltpu.sync_copy(agg_sh.at[rtail], agg_out.at[cid, rtail])

    @pl.when(sid == 0)
    def _():
        pltpu.sync_copy(den_sh, den_out.at[cid])


def _sc_layer(h, ei, s, t, zn):
    f = pl.kernel(
        _sc_layer_body,
        out_type=[jax.ShapeDtypeStruct((NC, N, D), jnp.float32),
                  jax.ShapeDtypeStruct((NC, N), jnp.float32),
                  jax.ShapeDtypeStruct((E,), jnp.float32)],
        mesh=_mesh,
        scratch_types=[
            pltpu.VMEM((CPT,), jnp.int32),
            pltpu.VMEM((NCH, CH), jnp.int32),
            pltpu.VMEM((CH,), jnp.float32),
            pltpu.VMEM((CH,), jnp.float32),
            pltpu.VMEM((CH,), jnp.float32),
            pltpu.VMEM((CH,), jnp.float32),
            pltpu.VMEM((CH,), jnp.float32),
            pltpu.VMEM((CH,), jnp.float32),
            pltpu.VMEM((CH, D), jnp.float32),
            pltpu.VMEM((CH, D), jnp.float32),
            pltpu.VMEM_SHARED((N, D), jnp.float32),
            pltpu.VMEM_SHARED((N,), jnp.float32),
        ] + [pltpu.SemaphoreType.DMA] * 12,
        compiler_params=_sc_params,
    )
    return f(h, ei, s, t, zn)


def _sc_loss_body(h_hbm, ei_hbm, w_hbm, den_hbm,
                  attn_out, d2_out,
                  src_v, dst_v, w_v, den0_v, den1_v, attn_v, d2_v,
                  ra0, ra1, rb0, rb1,
                  ga0, ga1, gb0, gb1):
    cid = lax.axis_index("c")
    sid = lax.axis_index("s")
    wid = cid * NS + sid
    base = wid * CPT

    rows_a = (ra0, ra1)
    rows_b = (rb0, rb1)
    gasem = (ga0, ga1)
    gbsem = (gb0, gb1)

    pltpu.sync_copy(den_hbm.at[0], den0_v)
    pltpu.sync_copy(den_hbm.at[1], den1_v)
    pltpu.sync_copy(w_hbm.at[pl.ds(base, CPT)], w_v)
    pltpu.sync_copy(ei_hbm.at[0, pl.ds(base, CPT)], src_v)
    pltpu.sync_copy(ei_hbm.at[1, pl.ds(base, CPT)], dst_v)

    def sbody(g, carry):
        sl = pl.ds(g * 16, 16)
        d16 = dst_v[sl]
        dg = plsc.load_gather(den0_v, [d16]) + plsc.load_gather(den1_v, [d16])
        attn_v[sl] = w_v[sl] / jnp.maximum(dg, 1e-16)
        return carry

    lax.fori_loop(0, CPT // 16, sbody, 0)
    pltpu.sync_copy(attn_v, attn_out.at[wid])

    def issue(jj, q):
        csl = pl.ds(jj * CH, CH)
        pltpu.async_copy(h_hbm.at[src_v.at[csl]], rows_a[q], gasem[q])
        pltpu.async_copy(h_hbm.at[dst_v.at[csl]], rows_b[q], gbsem[q])

    def step(j, p, issue_next):
        q = 1 - p
        if issue_next:
            @pl.when(j + 1 < NCH)
            def _():
                issue(j + 1, q)
        csl = pl.ds(j * CH, CH)
        pltpu.make_async_copy(h_hbm.at[src_v.at[csl]], rows_a[p],
                              gasem[p]).wait()
        pltpu.make_async_copy(h_hbm.at[dst_v.at[csl]], rows_b[p],
                              gbsem[p]).wait()

        def rbody(g2, c2):
            lanes = lax.iota(jnp.int32, 16)
            d2vec = jnp.zeros((16,), jnp.float32)
            for k in range(16):
                r = g2 * 16 + k
                acc = jnp.zeros((16,), jnp.float32)
                for u in range(D // 32):
                    qs = pl.ds(u * 32, 32)
                    dd = rows_a[p][r, qs] - rows_b[p][r, qs]
                    lo, hi = plsc.unpack(
                        dd, format=plsc.PackFormat.INTERLEAVED,
                        preferred_element_type=jnp.float32)
                    acc = acc + lo * lo + hi * hi
                d2vec = jnp.where(lanes == k, jnp.sum(acc), d2vec)
            d2_v[pl.ds(j * CH + g2 * 16, 16)] = d2vec
            return c2

        lax.fori_loop(0, CH // 16, rbody, 0)

    issue(0, 0)
    step(0, 0, issue_next=True)

    def pair(k2, carry):
        j0 = 2 * k2 + 1
        step(j0, 1, issue_next=True)
        step(j0 + 1, 0, issue_next=True)
        return carry

    lax.fori_loop(0, (NCH - 1) // 2, pair, 0)
    pltpu.sync_copy(d2_v, d2_out.at[wid])


def _sc_loss(h2, ei, w, den):
    f = pl.kernel(
        _sc_loss_body,
        out_type=[jax.ShapeDtypeStruct((NW, CPT), jnp.float32),
                  jax.ShapeDtypeStruct((NW, CPT), jnp.float32)],
        mesh=_mesh,
        scratch_types=[
            pltpu.VMEM((CPT,), jnp.int32),
            pltpu.VMEM((CPT,), jnp.int32),
            pltpu.VMEM((CPT,), jnp.float32),
            pltpu.VMEM((N,), jnp.float32),
            pltpu.VMEM((N,), jnp.float32),
            pltpu.VMEM((CPT,), jnp.float32),
            pltpu.VMEM((CPT,), jnp.float32),
            pltpu.VMEM((CH, D), jnp.bfloat16),
            pltpu.VMEM((CH, D), jnp.bfloat16),
            pltpu.VMEM((CH, D), jnp.bfloat16),
            pltpu.VMEM((CH, D), jnp.bfloat16),
        ] + [pltpu.SemaphoreType.DMA] * 4,
        compiler_params=_sc_params,
    )
    return f(h2, ei, w, den)


def kernel(embedding_input, edge_index, W_fc, a_attn):
    emb = embedding_input
    zn = jnp.zeros((N,), jnp.float32)

    s1, t1 = _tc_project(emb, W_fc, a_attn)
    agg1, den1, _ = _sc_layer(emb, edge_index, s1, t1, zn)
    h1, s2, t2 = _tc_combine_proj(emb, agg1, den1, W_fc, a_attn)

    agg2, den2, w2 = _sc_layer(h1, edge_index, s2, t2, zn)
    h2, h2b, la = _tc_combine_loss(emb, agg2, den2)

    attn, d2 = _sc_loss(h2b, edge_index, w2, den2)
    loss = _tc_loss_final(attn, d2, la)
    return h2, loss[0, 0]
